# baseline (speedup 1.0000x reference)
import functools

import jax
import jax.numpy as jnp
from jax import lax
from jax.experimental import pallas as pl
from jax.experimental.pallas import tpu as pltpu
from jax.experimental.pallas import tpu_sc as plsc

EMB = 64
NC = 2
NS = 16
NW = NC * NS
IDX_MINOR = 128
K = 4
CHUNK = K * IDX_MINOR


@functools.lru_cache(maxsize=None)
def _make_gather(n_total: int, vocab: int):
    assert n_total % (NW * CHUNK) == 0
    per_w = n_total // NW
    nchunks = per_w // CHUNK
    mesh = plsc.VectorSubcoreMesh(core_axis_name="c", subcore_axis_name="s")

    @functools.partial(
        pl.kernel,
        mesh=mesh,
        out_type=jax.ShapeDtypeStruct((NW * nchunks, CHUNK, EMB), jnp.float32),
        scratch_types=[
            pltpu.VMEM((K, IDX_MINOR), jnp.int32),
            pltpu.VMEM((CHUNK, EMB), jnp.float32),
            pltpu.SemaphoreType.DMA,
        ],
        compiler_params=pltpu.CompilerParams(use_tc_tiling_on_sc=False),
    )
    def gather_kernel(idx_hbm, table_hbm, out_hbm, idx_v, rows_v, sem):
        wid = lax.axis_index("s") * NC + lax.axis_index("c")
        base = wid * nchunks

        def chunk_body(i, carry):
            c = base + i
            pltpu.sync_copy(idx_hbm.at[c], idx_v)
            cps = [
                pltpu.async_copy(
                    table_hbm.at[idx_v.at[j]],
                    rows_v.at[pl.ds(j * IDX_MINOR, IDX_MINOR)],
                    sem,
                )
                for j in range(K)
            ]
            for cp in cps:
                cp.wait()
            pltpu.sync_copy(rows_v, out_hbm.at[c])
            return carry

        lax.fori_loop(0, nchunks, chunk_body, 0)

    return gather_kernel


def kernel(input, table):
    batch, hist = input.shape
    n_total = batch * hist
    idx = input.astype(jnp.int32).reshape(NW * (n_total // (NW * CHUNK)), K, IDX_MINOR)
    fn = _make_gather(n_total, table.shape[0])
    out = fn(idx, table)
    return out.reshape(batch, hist, EMB)


# speedup vs baseline: 1.0400x; 1.0400x over previous
import functools

import jax
import jax.numpy as jnp
from jax import lax
from jax.experimental import pallas as pl
from jax.experimental.pallas import tpu as pltpu
from jax.experimental.pallas import tpu_sc as plsc

EMB = 64
NC = 2
NS = 16
NW = NC * NS
IDX_MINOR = 128
K = 4
CHUNK = K * IDX_MINOR


@functools.lru_cache(maxsize=None)
def _make_gather(n_total: int, vocab: int):
    assert n_total % (NW * CHUNK) == 0
    per_w = n_total // NW
    nchunks = per_w // CHUNK
    assert nchunks % 2 == 0 and nchunks >= 4
    mesh = plsc.VectorSubcoreMesh(core_axis_name="c", subcore_axis_name="s")

    @functools.partial(
        pl.kernel,
        mesh=mesh,
        out_type=jax.ShapeDtypeStruct((NW * nchunks, CHUNK, EMB), jnp.float32),
        scratch_types=[
            pltpu.VMEM((2, K, IDX_MINOR), jnp.int32),
            pltpu.VMEM((2, CHUNK, EMB), jnp.float32),
            pltpu.SemaphoreType.DMA,
            pltpu.SemaphoreType.DMA,
            pltpu.SemaphoreType.DMA,
            pltpu.SemaphoreType.DMA,
            pltpu.SemaphoreType.DMA,
            pltpu.SemaphoreType.DMA,
        ],
        compiler_params=pltpu.CompilerParams(use_tc_tiling_on_sc=False),
    )
    def gather_kernel(idx_hbm, table_hbm, out_hbm, idx_v, rows_v,
                      si0, si1, sg0, sg1, so0, so1):
        wid = lax.axis_index("s") * NC + lax.axis_index("c")
        base = wid * nchunks
        sem_i = (si0, si1)
        sem_g = (sg0, sg1)
        sem_o = (so0, so1)

        def idx_cp(g, b):
            return pltpu.make_async_copy(idx_hbm.at[base + g], idx_v.at[b],
                                         sem_i[b])

        def gather_cps(b):
            return [
                pltpu.make_async_copy(
                    table_hbm.at[idx_v.at[b, j]],
                    rows_v.at[b, pl.ds(j * IDX_MINOR, IDX_MINOR)],
                    sem_g[b],
                )
                for j in range(K)
            ]

        def out_cp(g, b):
            return pltpu.make_async_copy(rows_v.at[b], out_hbm.at[base + g],
                                         sem_o[b])

        def start_gather(b):
            for cp in gather_cps(b):
                cp.start()

        def wait_gather(b):
            for cp in gather_cps(b):
                cp.wait()

        idx_cp(0, 0).start()
        idx_cp(0, 0).wait()
        start_gather(0)
        idx_cp(1, 1).start()

        wait_gather(0)
        out_cp(0, 0).start()
        idx_cp(2, 0).start()
        idx_cp(1, 1).wait()
        start_gather(1)

        wait_gather(1)
        out_cp(1, 1).start()
        idx_cp(3, 1).start()
        idx_cp(2, 0).wait()
        out_cp(0, 0).wait()
        start_gather(0)

        def pair_body(i, carry):
            u = 2 * i
            v = u + 1
            wait_gather(0)
            out_cp(u, 0).start()
            idx_cp(u + 2, 0).start()
            idx_cp(v, 1).wait()
            out_cp(v - 2, 1).wait()
            start_gather(1)

            wait_gather(1)
            out_cp(v, 1).start()
            idx_cp(v + 2, 1).start()
            idx_cp(u + 2, 0).wait()
            out_cp(u, 0).wait()
            start_gather(0)
            return carry

        lax.fori_loop(1, nchunks // 2 - 1, pair_body, 0)

        u = nchunks - 2
        v = nchunks - 1
        wait_gather(0)
        out_cp(u, 0).start()
        idx_cp(v, 1).wait()
        out_cp(v - 2, 1).wait()
        start_gather(1)

        wait_gather(1)
        out_cp(v, 1).start()
        out_cp(u, 0).wait()
        out_cp(v, 1).wait()

    return gather_kernel


def kernel(input, table):
    batch, hist = input.shape
    n_total = batch * hist
    idx = input.astype(jnp.int32).reshape(NW * (n_total // (NW * CHUNK)), K, IDX_MINOR)
    fn = _make_gather(n_total, table.shape[0])
    out = fn(idx, table)
    return out.reshape(batch, hist, EMB)
